# baseline (speedup 1.0000x reference)
import functools

import jax
import jax.numpy as jnp
from jax import lax
from jax.experimental import pallas as pl
from jax.experimental.pallas import tpu as pltpu
from jax.experimental.pallas import tpu_sc as plsc

_LANES = 16
_CHUNK = 128


def _sc_gather_call(n_rows, n_cols, n_table, num_workers, num_cores):
    cols_per_w = n_cols // num_workers
    n_chunks = cols_per_w // _CHUNK
    vecs_per_row = _CHUNK // _LANES
    mesh = plsc.VectorSubcoreMesh(core_axis_name="c", subcore_axis_name="s")

    @functools.partial(
        pl.kernel,
        mesh=mesh,
        out_type=jax.ShapeDtypeStruct((n_rows, n_cols), jnp.float32),
        scratch_types=[
            pltpu.VMEM((n_table,), jnp.float32),
            pltpu.VMEM((n_rows, _CHUNK), jnp.int32),
            pltpu.VMEM((n_rows, _CHUNK), jnp.int32),
            pltpu.VMEM((n_rows, _CHUNK), jnp.float32),
            pltpu.VMEM((n_rows, _CHUNK), jnp.float32),
            pltpu.SemaphoreType.DMA,
            pltpu.SemaphoreType.DMA,
            pltpu.SemaphoreType.DMA,
            pltpu.SemaphoreType.DMA,
        ],
        compiler_params=pltpu.CompilerParams(
            needs_layout_passes=False,
            disable_bounds_checks=True,
        ),
    )
    def sc_gather(
        table_hbm, idx_hbm, out_hbm,
        table_v, idx_v0, idx_v1, out_v0, out_v1, sin0, sin1, sout0, sout1,
    ):
        idx_bufs = (idx_v0, idx_v1)
        out_bufs = (out_v0, out_v1)
        sins = (sin0, sin1)
        souts = (sout0, sout1)
        wid = lax.axis_index("s") * num_cores + lax.axis_index("c")
        base = wid * cols_per_w

        in_cps = [None, None]
        out_cps = [None, None]
        in_cps[0] = pltpu.async_copy(
            idx_hbm.at[:, pl.ds(base, _CHUNK)], idx_bufs[0], sins[0]
        )
        if n_chunks > 1:
            in_cps[1] = pltpu.async_copy(
                idx_hbm.at[:, pl.ds(base + _CHUNK, _CHUNK)], idx_bufs[1], sins[1]
            )
        pltpu.sync_copy(table_hbm, table_v)
        for k in range(n_chunks):
            b = k & 1
            in_cps[b].wait()
            if out_cps[b] is not None:
                out_cps[b].wait()
            iv_buf = idx_bufs[b]
            ov_buf = out_bufs[b]

            @plsc.parallel_loop(0, n_rows, step=1, unroll=2)
            def body(r, iv_buf=iv_buf, ov_buf=ov_buf):
                for j in range(vecs_per_row):
                    c = j * _LANES
                    iv = iv_buf[r, pl.ds(c, _LANES)]
                    ov_buf[r, pl.ds(c, _LANES)] = plsc.load_gather(table_v, [iv])

            if k + 2 < n_chunks:
                in_cps[b] = pltpu.async_copy(
                    idx_hbm.at[:, pl.ds(base + (k + 2) * _CHUNK, _CHUNK)],
                    idx_bufs[b],
                    sins[b],
                )
            out_cps[b] = pltpu.async_copy(
                ov_buf, out_hbm.at[:, pl.ds(base + k * _CHUNK, _CHUNK)], souts[b]
            )
        for cp in out_cps:
            if cp is not None:
                cp.wait()

    return sc_gather


def kernel(propensities, position):
    n_rows, n_cols = position.shape
    idx_t = position.astype(jnp.int32).T
    table = propensities.astype(jnp.float32)

    info = plsc.get_sparse_core_info()
    num_workers = info.num_cores * info.num_subcores
    fn = _sc_gather_call(n_cols, n_rows, table.shape[0], num_workers, info.num_cores)
    return fn(table, idx_t).T


# speedup vs baseline: 1.0206x; 1.0206x over previous
import functools

import jax
import jax.numpy as jnp
from jax import lax
from jax.experimental import pallas as pl
from jax.experimental.pallas import tpu as pltpu
from jax.experimental.pallas import tpu_sc as plsc

_LANES = 16
_CHUNK = 256


def _sc_gather_call(n_rows, n_cols, n_table, num_workers, num_cores):
    cols_per_w = n_cols // num_workers
    n_chunks = cols_per_w // _CHUNK
    vecs_per_row = _CHUNK // _LANES
    mesh = plsc.VectorSubcoreMesh(core_axis_name="c", subcore_axis_name="s")

    @functools.partial(
        pl.kernel,
        mesh=mesh,
        out_type=jax.ShapeDtypeStruct((n_rows, n_cols), jnp.float32),
        scratch_types=[
            pltpu.VMEM((n_table,), jnp.float32),
            pltpu.VMEM((n_rows, _CHUNK), jnp.int32),
            pltpu.VMEM((n_rows, _CHUNK), jnp.int32),
            pltpu.VMEM((n_rows, _CHUNK), jnp.float32),
            pltpu.VMEM((n_rows, _CHUNK), jnp.float32),
            pltpu.SemaphoreType.DMA,
            pltpu.SemaphoreType.DMA,
            pltpu.SemaphoreType.DMA,
            pltpu.SemaphoreType.DMA,
        ],
        compiler_params=pltpu.CompilerParams(
            needs_layout_passes=False,
            disable_bounds_checks=True,
        ),
    )
    def sc_gather(
        table_hbm, idx_hbm, out_hbm,
        table_v, idx_v0, idx_v1, out_v0, out_v1, sin0, sin1, sout0, sout1,
    ):
        idx_bufs = (idx_v0, idx_v1)
        out_bufs = (out_v0, out_v1)
        sins = (sin0, sin1)
        souts = (sout0, sout1)
        wid = lax.axis_index("s") * num_cores + lax.axis_index("c")
        base = wid * cols_per_w

        in_cps = [None, None]
        out_cps = [None, None]
        in_cps[0] = pltpu.async_copy(
            idx_hbm.at[:, pl.ds(base, _CHUNK)], idx_bufs[0], sins[0]
        )
        if n_chunks > 1:
            in_cps[1] = pltpu.async_copy(
                idx_hbm.at[:, pl.ds(base + _CHUNK, _CHUNK)], idx_bufs[1], sins[1]
            )
        pltpu.sync_copy(table_hbm, table_v)
        for k in range(n_chunks):
            b = k & 1
            in_cps[b].wait()
            if out_cps[b] is not None:
                out_cps[b].wait()
            iv_buf = idx_bufs[b]
            ov_buf = out_bufs[b]

            @plsc.parallel_loop(0, n_rows, step=1, unroll=2)
            def body(r, iv_buf=iv_buf, ov_buf=ov_buf):
                for j in range(vecs_per_row):
                    c = j * _LANES
                    iv = iv_buf[r, pl.ds(c, _LANES)]
                    ov_buf[r, pl.ds(c, _LANES)] = plsc.load_gather(table_v, [iv])

            if k + 2 < n_chunks:
                in_cps[b] = pltpu.async_copy(
                    idx_hbm.at[:, pl.ds(base + (k + 2) * _CHUNK, _CHUNK)],
                    idx_bufs[b],
                    sins[b],
                )
            out_cps[b] = pltpu.async_copy(
                ov_buf, out_hbm.at[:, pl.ds(base + k * _CHUNK, _CHUNK)], souts[b]
            )
        for cp in out_cps:
            if cp is not None:
                cp.wait()

    return sc_gather


def kernel(propensities, position):
    n_rows, n_cols = position.shape
    idx_t = position.astype(jnp.int32).T
    table = propensities.astype(jnp.float32)

    info = plsc.get_sparse_core_info()
    num_workers = info.num_cores * info.num_subcores
    fn = _sc_gather_call(n_cols, n_rows, table.shape[0], num_workers, info.num_cores)
    return fn(table, idx_t).T


# speedup vs baseline: 1.0311x; 1.0102x over previous
import functools

import jax
import jax.numpy as jnp
from jax import lax
from jax.experimental import pallas as pl
from jax.experimental.pallas import tpu as pltpu
from jax.experimental.pallas import tpu_sc as plsc

_LANES = 16
_CHUNKS = (128, 256, 128)
_CHUNK_MAX = max(_CHUNKS)


def _sc_gather_call(n_rows, n_cols, n_table, num_workers, num_cores):
    cols_per_w = n_cols // num_workers
    assert cols_per_w == sum(_CHUNKS)
    n_chunks = len(_CHUNKS)
    chunk_off = [sum(_CHUNKS[:i]) for i in range(n_chunks)]
    mesh = plsc.VectorSubcoreMesh(core_axis_name="c", subcore_axis_name="s")

    @functools.partial(
        pl.kernel,
        mesh=mesh,
        out_type=jax.ShapeDtypeStruct((n_rows, n_cols), jnp.float32),
        scratch_types=[
            pltpu.VMEM((n_table,), jnp.float32),
            pltpu.VMEM((n_rows, _CHUNK_MAX), jnp.int32),
            pltpu.VMEM((n_rows, _CHUNK_MAX), jnp.int32),
            pltpu.VMEM((n_rows, _CHUNK_MAX), jnp.float32),
            pltpu.VMEM((n_rows, _CHUNK_MAX), jnp.float32),
            pltpu.SemaphoreType.DMA,
            pltpu.SemaphoreType.DMA,
            pltpu.SemaphoreType.DMA,
            pltpu.SemaphoreType.DMA,
        ],
        compiler_params=pltpu.CompilerParams(
            needs_layout_passes=False,
            disable_bounds_checks=True,
        ),
    )
    def sc_gather(
        table_hbm, idx_hbm, out_hbm,
        table_v, idx_v0, idx_v1, out_v0, out_v1, sin0, sin1, sout0, sout1,
    ):
        idx_bufs = (idx_v0, idx_v1)
        out_bufs = (out_v0, out_v1)
        sins = (sin0, sin1)
        souts = (sout0, sout1)
        wid = lax.axis_index("s") * num_cores + lax.axis_index("c")
        base = wid * cols_per_w

        def copy_in(k, b):
            sz = _CHUNKS[k]
            return pltpu.async_copy(
                idx_hbm.at[:, pl.ds(base + chunk_off[k], sz)],
                idx_bufs[b].at[:, pl.ds(0, sz)],
                sins[b],
            )

        in_cps = [None, None]
        out_cps = [None, None]
        in_cps[0] = copy_in(0, 0)
        if n_chunks > 1:
            in_cps[1] = copy_in(1, 1)
        pltpu.sync_copy(table_hbm, table_v)
        for k in range(n_chunks):
            b = k & 1
            sz = _CHUNKS[k]
            in_cps[b].wait()
            if out_cps[b] is not None:
                out_cps[b].wait()
            iv_buf = idx_bufs[b]
            ov_buf = out_bufs[b]

            @plsc.parallel_loop(0, n_rows, step=1, unroll=2)
            def body(r, iv_buf=iv_buf, ov_buf=ov_buf, sz=sz):
                for j in range(sz // _LANES):
                    c = j * _LANES
                    iv = iv_buf[r, pl.ds(c, _LANES)]
                    ov_buf[r, pl.ds(c, _LANES)] = plsc.load_gather(table_v, [iv])

            if k + 2 < n_chunks:
                in_cps[b] = copy_in(k + 2, b)
            out_cps[b] = pltpu.async_copy(
                ov_buf.at[:, pl.ds(0, sz)],
                out_hbm.at[:, pl.ds(base + chunk_off[k], sz)],
                souts[b],
            )
        for cp in out_cps:
            if cp is not None:
                cp.wait()

    return sc_gather


def kernel(propensities, position):
    n_rows, n_cols = position.shape
    idx_t = position.astype(jnp.int32).T
    table = propensities.astype(jnp.float32)

    info = plsc.get_sparse_core_info()
    num_workers = info.num_cores * info.num_subcores
    fn = _sc_gather_call(n_cols, n_rows, table.shape[0], num_workers, info.num_cores)
    return fn(table, idx_t).T
